# baseline (speedup 1.0000x reference)
import functools

import jax
import jax.numpy as jnp
from jax.experimental import pallas as pl
from jax.experimental.pallas import tpu as pltpu

_TM = 256
_TE = 512
_KBLK = 1024


def _dotT(a, b):
    return jax.lax.dot_general(a, b, (((0,), (0,)), ((), ())),
                               preferred_element_type=jnp.float32)


def _mm_body(a_ref, b_ref, *rest, bias, addend, act):
    idx = 0
    refs = list(rest)
    acc = jnp.dot(a_ref[...], b_ref[...], preferred_element_type=jnp.float32)
    if bias:
        acc = acc + refs[idx][0:1, :]
        idx += 1
    if addend:
        acc = acc + refs[idx][...]
        idx += 1
    if act == "elu":
        acc = jnp.where(acc > 0, acc, jnp.exp(jnp.minimum(acc, 0.0)) - 1.0)
    refs[idx][...] = acc


def _mm(a, b, bias=None, addend=None, act=None):
    m, k = a.shape
    _, c = b.shape
    grid = (m // _TM, c // 128)
    in_specs = [
        pl.BlockSpec((_TM, k), lambda i, j: (i, 0)),
        pl.BlockSpec((k, 128), lambda i, j: (0, j)),
    ]
    args = [a, b]
    if bias is not None:
        args.append(jnp.broadcast_to(bias[None, :], (8, c)))
        in_specs.append(pl.BlockSpec((8, 128), lambda i, j: (0, j)))
    if addend is not None:
        args.append(addend)
        in_specs.append(pl.BlockSpec((_TM, 128), lambda i, j: (i, j)))
    body = functools.partial(_mm_body, bias=bias is not None,
                             addend=addend is not None, act=act)
    return pl.pallas_call(
        body, grid=grid, in_specs=in_specs,
        out_specs=pl.BlockSpec((_TM, 128), lambda i, j: (i, j)),
        out_shape=jax.ShapeDtypeStruct((m, c), jnp.float32),
    )(*args)


def _gather_body(src_ref, dst_ref, h_ref, es_ref, ed_ref, r_ref,
                 prod_ref, ez_ref, hs, ess, eds, *, kb_n):
    kb = pl.program_id(1)

    @pl.when(kb == 0)
    def _():
        hs[...] = jnp.zeros_like(hs)
        ess[...] = jnp.zeros_like(ess)
        eds[...] = jnp.zeros_like(eds)

    rows = jax.lax.broadcasted_iota(jnp.int32, (_KBLK, _TE), 0) + kb * _KBLK
    ohs = (rows == jnp.broadcast_to(src_ref[0], (_KBLK, _TE))
           ).astype(jnp.float32)
    ohd = (rows == jnp.broadcast_to(dst_ref[0], (_KBLK, _TE))
           ).astype(jnp.float32)
    hs[...] += _dotT(ohs, h_ref[...])
    ess[...] += _dotT(ohs, es_ref[...])
    eds[...] += _dotT(ohd, ed_ref[...])

    @pl.when(kb == kb_n - 1)
    def _():
        e = ess[...] + eds[...]
        e = jnp.where(e >= 0, e, 0.2 * e)
        ez = jnp.exp(e)
        ez_ref[...] = ez
        prod_ref[...] = hs[...] * jnp.dot(
            ez, r_ref[...], preferred_element_type=jnp.float32)


def _gather(h, es, ed, src_l, dst_l, r):
    npad, ch = h.shape
    eb_n = src_l.shape[0]
    kb_n = npad // _KBLK
    epad = eb_n * _TE
    body = functools.partial(_gather_body, kb_n=kb_n)
    return pl.pallas_call(
        body, grid=(eb_n, kb_n),
        in_specs=[
            pl.BlockSpec((1, 1, _TE), lambda e, k: (e, 0, 0)),
            pl.BlockSpec((1, 1, _TE), lambda e, k: (e, 0, 0)),
            pl.BlockSpec((_KBLK, ch), lambda e, k: (k, 0)),
            pl.BlockSpec((_KBLK, 128), lambda e, k: (k, 0)),
            pl.BlockSpec((_KBLK, 128), lambda e, k: (k, 0)),
            pl.BlockSpec((128, ch), lambda e, k: (0, 0)),
        ],
        out_specs=[
            pl.BlockSpec((_TE, ch), lambda e, k: (e, 0)),
            pl.BlockSpec((_TE, 128), lambda e, k: (e, 0)),
        ],
        out_shape=[
            jax.ShapeDtypeStruct((epad, ch), jnp.float32),
            jax.ShapeDtypeStruct((epad, 128), jnp.float32),
        ],
        scratch_shapes=[
            pltpu.VMEM((_TE, ch), jnp.float32),
            pltpu.VMEM((_TE, 128), jnp.float32),
            pltpu.VMEM((_TE, 128), jnp.float32),
        ],
    )(src_l, dst_l, h, es, ed, r)


def _scatter_body(dst_ref, prod_ref, ez_ref, r_ref, *rest, eb_n, mean):
    if mean:
        rm_ref, out_ref, num, den = rest
    else:
        out_ref, num, den = rest
    nt = pl.program_id(0)
    eb = pl.program_id(1)

    @pl.when(eb == 0)
    def _():
        num[...] = jnp.zeros_like(num)
        den[...] = jnp.zeros_like(den)

    cols = jax.lax.broadcasted_iota(jnp.int32, (_TE, _TM), 1) + nt * _TM
    ohd = (jnp.broadcast_to(dst_ref[0], (_TE, _TM)) == cols
           ).astype(jnp.float32)
    num[...] += _dotT(ohd, prod_ref[...])
    den[...] += _dotT(ohd, ez_ref[...])

    @pl.when(eb == eb_n - 1)
    def _():
        denb = jnp.dot(den[...], r_ref[...],
                       preferred_element_type=jnp.float32) + 1e-16
        o = num[...] / denb
        if mean:
            o = jnp.dot(o, rm_ref[...], preferred_element_type=jnp.float32)
        out_ref[...] = o


def _scatter(prod, ez, dst_c, r, rm, npad):
    epad, ch = prod.shape
    eb_n = epad // _TE
    cout = ch if rm is None else rm.shape[1]
    body = functools.partial(_scatter_body, eb_n=eb_n, mean=rm is not None)
    in_specs = [
        pl.BlockSpec((1, _TE, 1), lambda n, e: (e, 0, 0)),
        pl.BlockSpec((_TE, ch), lambda n, e: (e, 0)),
        pl.BlockSpec((_TE, 128), lambda n, e: (e, 0)),
        pl.BlockSpec((128, ch), lambda n, e: (0, 0)),
    ]
    args = [dst_c, prod, ez, r]
    if rm is not None:
        in_specs.append(pl.BlockSpec((ch, cout), lambda n, e: (0, 0)))
        args.append(rm)
    return pl.pallas_call(
        body, grid=(npad // _TM, eb_n),
        in_specs=in_specs,
        out_specs=pl.BlockSpec((_TM, cout), lambda n, e: (n, 0)),
        out_shape=jax.ShapeDtypeStruct((npad, cout), jnp.float32),
        scratch_shapes=[
            pltpu.VMEM((_TM, ch), jnp.float32),
            pltpu.VMEM((_TM, 128), jnp.float32),
        ],
    )(*args)


def _att_mat(a, ch):
    heads, chead = a.shape
    m = (jnp.eye(heads, dtype=jnp.float32)[:, None, :]
         * a[:, :, None]).reshape(heads * chead, heads)
    return jnp.pad(m, ((0, ch - heads * chead), (0, 128 - heads)))


def _rep_mat(heads, chead, ch):
    r = jnp.kron(jnp.eye(heads, dtype=jnp.float32), jnp.ones((1, chead), jnp.float32))
    return jnp.pad(r, ((0, 128 - heads), (0, ch - heads * chead)))


def _gat_layer(x, h, a_s, a_d, src_l, dst_l, dst_c, npad, heads, chead, mean):
    ch = h.shape[1]
    es = _mm(h, _att_mat(a_s, ch))
    ed = _mm(h, _att_mat(a_d, ch))
    r = _rep_mat(heads, chead, ch)
    prod, ez = _gather(h, es, ed, src_l, dst_l, r)
    rm = None
    if mean:
        rm = jnp.tile(jnp.eye(128, dtype=jnp.float32), (heads, 1)) / heads
    return _scatter(prod, ez, dst_c, r, rm, npad)


def kernel(x, edge_index, W_map, b_map, W1, att_src1, att_dst1, b1, Wl1, bl1,
           W2, att_src2, att_dst2, b2, Wl2, bl2,
           W3, att_src3, att_dst3, b3, Wl3, bl3):
    n, _ = x.shape
    c = b3.shape[0]
    npad = ((n + _KBLK - 1) // _KBLK) * _KBLK
    src = jnp.concatenate([edge_index[0], jnp.arange(n, dtype=edge_index.dtype)])
    dst = jnp.concatenate([edge_index[1], jnp.arange(n, dtype=edge_index.dtype)])
    e_tot = src.shape[0]
    epad = ((e_tot + _TE - 1) // _TE) * _TE
    src = jnp.pad(src, (0, epad - e_tot), constant_values=npad - 1).astype(jnp.int32)
    dst = jnp.pad(dst, (0, epad - e_tot), constant_values=npad - 1).astype(jnp.int32)
    eb_n = epad // _TE
    src_l = src.reshape(eb_n, 1, _TE)
    dst_l = dst.reshape(eb_n, 1, _TE)
    dst_c = dst.reshape(eb_n, _TE, 1)

    xp = jnp.pad(x, ((0, npad - n), (0, 0)))
    x0 = _mm(xp, W_map, bias=b_map)

    h1 = _mm(x0, W1)
    g1 = _gat_layer(x0, h1, att_src1, att_dst1, src_l, dst_l, dst_c,
                    npad, 4, 256, False)
    x1 = _mm(x0, Wl1, bias=b1 + bl1, addend=g1, act="elu")

    h2 = _mm(x1, W2)
    g2 = _gat_layer(x1, h2, att_src2, att_dst2, src_l, dst_l, dst_c,
                    npad, 4, 256, False)
    x2 = _mm(x1, Wl2, bias=b2 + bl2, addend=g2, act="elu")

    w3p = jnp.pad(W3.reshape(-1, 6, c), ((0, 0), (0, 0), (0, 128 - c))
                  ).reshape(-1, 6 * 128)
    as3 = jnp.pad(att_src3, ((0, 0), (0, 128 - c)))
    ad3 = jnp.pad(att_dst3, ((0, 0), (0, 128 - c)))
    h3 = _mm(x2, w3p)
    g3 = _gat_layer(x2, h3, as3, ad3, src_l, dst_l, dst_c,
                    npad, 6, 128, True)
    wl3p = jnp.pad(Wl3, ((0, 0), (0, 128 - c)))
    b3p = jnp.pad(b3 + bl3, (0, 128 - c))
    x3 = _mm(x2, wl3p, bias=b3p, addend=g3)
    return x3[:n, :c]


# speedup vs baseline: 1.1430x; 1.1430x over previous
import functools

import jax
import jax.numpy as jnp
from jax.experimental import pallas as pl
from jax.experimental.pallas import tpu as pltpu

_TM = 256
_TE = 512
_KBLK = 1024


def _dotT(a, b):
    return jax.lax.dot_general(a, b, (((0,), (0,)), ((), ())),
                               preferred_element_type=jnp.float32)


def _mm_body(a_ref, b_ref, *rest, bias, addend, act):
    idx = 0
    refs = list(rest)
    acc = jnp.dot(a_ref[...], b_ref[...], preferred_element_type=jnp.float32)
    if bias:
        acc = acc + refs[idx][0:1, :]
        idx += 1
    if addend:
        acc = acc + refs[idx][...]
        idx += 1
    if act == "elu":
        acc = jnp.where(acc > 0, acc, jnp.exp(jnp.minimum(acc, 0.0)) - 1.0)
    refs[idx][...] = acc


def _mm(a, b, bias=None, addend=None, act=None):
    m, k = a.shape
    _, c = b.shape
    grid = (m // _TM, c // 128)
    in_specs = [
        pl.BlockSpec((_TM, k), lambda i, j: (i, 0)),
        pl.BlockSpec((k, 128), lambda i, j: (0, j)),
    ]
    args = [a, b]
    if bias is not None:
        args.append(jnp.broadcast_to(bias[None, :], (8, c)))
        in_specs.append(pl.BlockSpec((8, 128), lambda i, j: (0, j)))
    if addend is not None:
        args.append(addend)
        in_specs.append(pl.BlockSpec((_TM, 128), lambda i, j: (i, j)))
    body = functools.partial(_mm_body, bias=bias is not None,
                             addend=addend is not None, act=act)
    return pl.pallas_call(
        body, grid=grid, in_specs=in_specs,
        out_specs=pl.BlockSpec((_TM, 128), lambda i, j: (i, j)),
        out_shape=jax.ShapeDtypeStruct((m, c), jnp.float32),
    )(*args)


def _gather_body(src_ref, dst_ref, h_ref, es_ref, ed_ref, r_ref,
                 prod_ref, ez_ref, hs, ess, eds, *, kb_n):
    kb = pl.program_id(1)

    @pl.when(kb == 0)
    def _():
        hs[...] = jnp.zeros_like(hs)
        ess[...] = jnp.zeros_like(ess)
        eds[...] = jnp.zeros_like(eds)

    rows = jax.lax.broadcasted_iota(jnp.int32, (_KBLK, _TE), 0) + kb * _KBLK
    ohs = (rows == jnp.broadcast_to(src_ref[0], (_KBLK, _TE))
           ).astype(jnp.bfloat16)
    ohd = (rows == jnp.broadcast_to(dst_ref[0], (_KBLK, _TE))
           ).astype(jnp.bfloat16)
    hs[...] += _dotT(ohs, h_ref[...].astype(jnp.bfloat16))
    ess[...] += _dotT(ohs, es_ref[...].astype(jnp.bfloat16))
    eds[...] += _dotT(ohd, ed_ref[...].astype(jnp.bfloat16))

    @pl.when(kb == kb_n - 1)
    def _():
        e = ess[...] + eds[...]
        e = jnp.where(e >= 0, e, 0.2 * e)
        ez = jnp.exp(e)
        ez_ref[...] = ez.astype(jnp.bfloat16)
        prod_ref[...] = (hs[...] * jnp.dot(
            ez, r_ref[...], preferred_element_type=jnp.float32)
                         ).astype(jnp.bfloat16)


def _gather(h, es, ed, src_l, dst_l, r):
    npad, ch = h.shape
    eb_n = src_l.shape[0]
    kb_n = npad // _KBLK
    epad = eb_n * _TE
    body = functools.partial(_gather_body, kb_n=kb_n)
    return pl.pallas_call(
        body, grid=(eb_n, kb_n),
        in_specs=[
            pl.BlockSpec((1, 1, _TE), lambda e, k: (e, 0, 0)),
            pl.BlockSpec((1, 1, _TE), lambda e, k: (e, 0, 0)),
            pl.BlockSpec((_KBLK, ch), lambda e, k: (k, 0)),
            pl.BlockSpec((_KBLK, 128), lambda e, k: (k, 0)),
            pl.BlockSpec((_KBLK, 128), lambda e, k: (k, 0)),
            pl.BlockSpec((128, ch), lambda e, k: (0, 0)),
        ],
        out_specs=[
            pl.BlockSpec((_TE, ch), lambda e, k: (e, 0)),
            pl.BlockSpec((_TE, 128), lambda e, k: (e, 0)),
        ],
        out_shape=[
            jax.ShapeDtypeStruct((epad, ch), jnp.bfloat16),
            jax.ShapeDtypeStruct((epad, 128), jnp.bfloat16),
        ],
        scratch_shapes=[
            pltpu.VMEM((_TE, ch), jnp.float32),
            pltpu.VMEM((_TE, 128), jnp.float32),
            pltpu.VMEM((_TE, 128), jnp.float32),
        ],
    )(src_l, dst_l, h, es, ed, r)


def _scatter_body(dst_ref, prod_ref, ez_ref, r_ref, *rest, eb_n, mean):
    if mean:
        rm_ref, out_ref, num, den = rest
    else:
        out_ref, num, den = rest
    nt = pl.program_id(0)
    eb = pl.program_id(1)

    @pl.when(eb == 0)
    def _():
        num[...] = jnp.zeros_like(num)
        den[...] = jnp.zeros_like(den)

    cols = jax.lax.broadcasted_iota(jnp.int32, (_TE, _TM), 1) + nt * _TM
    ohd = (jnp.broadcast_to(dst_ref[0], (_TE, _TM)) == cols
           ).astype(jnp.bfloat16)
    num[...] += _dotT(ohd, prod_ref[...])
    den[...] += _dotT(ohd, ez_ref[...])

    @pl.when(eb == eb_n - 1)
    def _():
        denb = jnp.dot(den[...], r_ref[...],
                       preferred_element_type=jnp.float32) + 1e-16
        o = num[...] / denb
        if mean:
            o = jnp.dot(o, rm_ref[...], preferred_element_type=jnp.float32)
        out_ref[...] = o


def _scatter(prod, ez, dst_c, r, rm, npad):
    epad, ch = prod.shape
    eb_n = epad // _TE
    cout = ch if rm is None else rm.shape[1]
    body = functools.partial(_scatter_body, eb_n=eb_n, mean=rm is not None)
    in_specs = [
        pl.BlockSpec((1, _TE, 1), lambda n, e: (e, 0, 0)),
        pl.BlockSpec((_TE, ch), lambda n, e: (e, 0)),
        pl.BlockSpec((_TE, 128), lambda n, e: (e, 0)),
        pl.BlockSpec((128, ch), lambda n, e: (0, 0)),
    ]
    args = [dst_c, prod, ez, r]
    if rm is not None:
        in_specs.append(pl.BlockSpec((ch, cout), lambda n, e: (0, 0)))
        args.append(rm)
    return pl.pallas_call(
        body, grid=(npad // _TM, eb_n),
        in_specs=in_specs,
        out_specs=pl.BlockSpec((_TM, cout), lambda n, e: (n, 0)),
        out_shape=jax.ShapeDtypeStruct((npad, cout), jnp.float32),
        scratch_shapes=[
            pltpu.VMEM((_TM, ch), jnp.float32),
            pltpu.VMEM((_TM, 128), jnp.float32),
        ],
    )(*args)


def _att_mat(a, ch):
    heads, chead = a.shape
    m = (jnp.eye(heads, dtype=jnp.float32)[:, None, :]
         * a[:, :, None]).reshape(heads * chead, heads)
    return jnp.pad(m, ((0, ch - heads * chead), (0, 128 - heads)))


def _rep_mat(heads, chead, ch):
    r = jnp.kron(jnp.eye(heads, dtype=jnp.float32), jnp.ones((1, chead), jnp.float32))
    return jnp.pad(r, ((0, 128 - heads), (0, ch - heads * chead)))


def _gat_layer(x, h, a_s, a_d, src_l, dst_l, dst_c, npad, heads, chead, mean):
    ch = h.shape[1]
    es = _mm(h, _att_mat(a_s, ch))
    ed = _mm(h, _att_mat(a_d, ch))
    r = _rep_mat(heads, chead, ch)
    prod, ez = _gather(h, es, ed, src_l, dst_l, r)
    rm = None
    if mean:
        rm = jnp.tile(jnp.eye(128, dtype=jnp.float32), (heads, 1)) / heads
    return _scatter(prod, ez, dst_c, r, rm, npad)


def kernel(x, edge_index, W_map, b_map, W1, att_src1, att_dst1, b1, Wl1, bl1,
           W2, att_src2, att_dst2, b2, Wl2, bl2,
           W3, att_src3, att_dst3, b3, Wl3, bl3):
    n, _ = x.shape
    c = b3.shape[0]
    npad = ((n + _KBLK - 1) // _KBLK) * _KBLK
    src = jnp.concatenate([edge_index[0], jnp.arange(n, dtype=edge_index.dtype)])
    dst = jnp.concatenate([edge_index[1], jnp.arange(n, dtype=edge_index.dtype)])
    e_tot = src.shape[0]
    epad = ((e_tot + _TE - 1) // _TE) * _TE
    src = jnp.pad(src, (0, epad - e_tot), constant_values=npad - 1).astype(jnp.int32)
    dst = jnp.pad(dst, (0, epad - e_tot), constant_values=npad - 1).astype(jnp.int32)
    eb_n = epad // _TE
    src_l = src.reshape(eb_n, 1, _TE)
    dst_l = dst.reshape(eb_n, 1, _TE)
    dst_c = dst.reshape(eb_n, _TE, 1)

    xp = jnp.pad(x, ((0, npad - n), (0, 0)))
    x0 = _mm(xp, W_map, bias=b_map)

    h1 = _mm(x0, W1)
    g1 = _gat_layer(x0, h1, att_src1, att_dst1, src_l, dst_l, dst_c,
                    npad, 4, 256, False)
    x1 = _mm(x0, Wl1, bias=b1 + bl1, addend=g1, act="elu")

    h2 = _mm(x1, W2)
    g2 = _gat_layer(x1, h2, att_src2, att_dst2, src_l, dst_l, dst_c,
                    npad, 4, 256, False)
    x2 = _mm(x1, Wl2, bias=b2 + bl2, addend=g2, act="elu")

    w3p = jnp.pad(W3.reshape(-1, 6, c), ((0, 0), (0, 0), (0, 128 - c))
                  ).reshape(-1, 6 * 128)
    as3 = jnp.pad(att_src3, ((0, 0), (0, 128 - c)))
    ad3 = jnp.pad(att_dst3, ((0, 0), (0, 128 - c)))
    h3 = _mm(x2, w3p)
    g3 = _gat_layer(x2, h3, as3, ad3, src_l, dst_l, dst_c,
                    npad, 6, 128, True)
    wl3p = jnp.pad(Wl3, ((0, 0), (0, 128 - c)))
    b3p = jnp.pad(b3 + bl3, (0, 128 - c))
    x3 = _mm(x2, wl3p, bias=b3p, addend=g3)
    return x3[:n, :c]
